# baseline (speedup 1.0000x reference)
import functools

import jax
import jax.numpy as jnp
from jax import lax
from jax.experimental import pallas as pl
from jax.experimental.pallas import tpu as pltpu
from jax.experimental.pallas import tpu_sc as plsc

NUM_PLAYERS = 50000
NUM_TEAMS = 2000
FDIM = 64
NUM_GRAPHS = 64
NUM_EDGES = 800000

NCORES = 2
NSUB = 16

KB = 16
B0 = 64
B1 = 128

EDGES_PAD = 819200
NSB0 = EDGES_PAD // B0
NSB1 = EDGES_PAD // B1
NBLK_A = NSB0 // (NSUB * KB)
PB_PER_TILE = NSB0 // NCORES // NSUB
NBLK_B = PB_PER_TILE // KB
KB_B = 8
L1_PER_TILE = NSB1 // NCORES // NSUB
NBLK_L1 = L1_PER_TILE // KB_B

P_HALF = NUM_PLAYERS // NCORES
PAD = 40
DUMMY_MASK = 31
CHUNK = 40
AGGP_ROWS = P_HALF + PAD
AGGT_ROWS = NUM_TEAMS + PAD
NPCHUNK = P_HALF // CHUNK
NZPCHUNK = AGGP_ROWS // CHUNK
NTCHUNK = NUM_TEAMS // CHUNK
NZTCHUNK = AGGT_ROWS // CHUNK


def _zero_zbuf(zbuf):
    zero16 = jnp.zeros((16,), jnp.float32)

    def zrow(i, _):
        for j in range(FDIM // 16):
            zbuf[i, pl.ds(j * 16, 16)] = zero16
        return 0

    lax.fori_loop(0, CHUNK, zrow, 0)


def _strided_chunks(sid, nchunk, body):
    def it(i, _):
        c = sid + i * NSUB

        @pl.when(c < nchunk)
        def _():
            body(c)

        return 0

    lax.fori_loop(0, (nchunk + NSUB - 1) // NSUB, it, 0)


def _run_pipe(kb, table_at, gidx, tgt, sidx, rows, gsem, ssem):
    depth = len(rows)
    look = depth - 1
    g = [None] * depth
    pend = [None] * depth
    for j in range(kb + look):
        if j < kb:
            p = j % depth
            if pend[p] is not None:
                pend[p].wait()
                pend[p] = None
            g[p] = pltpu.async_copy(table_at(gidx.at[j]), rows[p], gsem[p])
        if j >= look:
            jj = j - look
            q = jj % depth
            g[q].wait()
            pend[q] = pltpu.async_copy(rows[q], tgt.at[sidx.at[jj]],
                                       ssem[q], add=True)
    for q in range(depth):
        if pend[q] is not None:
            pend[q].wait()


def _sc_layer0(src2, dst2, x_p, x_t):
    mesh = plsc.VectorSubcoreMesh(core_axis_name="c", subcore_axis_name="s")

    @functools.partial(
        pl.kernel,
        mesh=mesh,
        compiler_params=pltpu.CompilerParams(use_tc_tiling_on_sc=False),
        out_type=[
            jax.ShapeDtypeStruct((NCORES, NUM_TEAMS, FDIM), jnp.float32),
            jax.ShapeDtypeStruct((NUM_PLAYERS, FDIM), jnp.float32),
        ],
        scratch_types=(
            [pltpu.VMEM_SHARED((AGGP_ROWS, FDIM), jnp.float32),
             pltpu.VMEM_SHARED((AGGT_ROWS, FDIM), jnp.float32),
             pltpu.VMEM((KB, B0), jnp.int32),
             pltpu.VMEM((KB, B0), jnp.int32)]
            + [pltpu.VMEM((B0, FDIM), jnp.float32) for _ in range(4)]
            + [pltpu.VMEM((CHUNK, FDIM), jnp.float32)]
            + [pltpu.SemaphoreType.DMA for _ in range(8)]
        ),
    )
    def k(src_h, dst_h, xp_h, xt_h, aggt_out, aggp_out,
          aggp_s, aggt_s, gi_v, si_v,
          r0, r1, r2, r3, zbuf, g0, g1, g2, g3, s0, s1, s2, s3):
        cid = lax.axis_index("c")
        sid = lax.axis_index("s")
        lo = cid * P_HALF
        rows = [r0, r1, r2, r3]
        gsem = [g0, g1, g2, g3]
        ssem = [s0, s1, s2, s3]

        _zero_zbuf(zbuf)
        _strided_chunks(
            sid, NZTCHUNK,
            lambda c: pltpu.sync_copy(zbuf,
                                      aggt_s.at[pl.ds(c * CHUNK, CHUNK)]))
        _strided_chunks(
            sid, NZPCHUNK,
            lambda c: pltpu.sync_copy(zbuf,
                                      aggp_s.at[pl.ds(c * CHUNK, CHUNK)]))
        plsc.subcore_barrier()

        iota16 = lax.iota(jnp.int32, 16)

        a_start = sid * (NBLK_A * KB)

        def blk_a(bi, _):
            base = a_start + bi * KB
            pltpu.sync_copy(dst_h.at[pl.ds(base, KB)], gi_v)
            pltpu.sync_copy(src_h.at[pl.ds(base, KB)], si_v)
            for j in range(KB):
                for q in range(B0 // 16):
                    v = si_v[j, pl.ds(q * 16, 16)]
                    owned = (v >= lo) & (v < lo + P_HALF)
                    dummy = P_HALF + ((iota16 + q * 16) & DUMMY_MASK)
                    si_v[j, pl.ds(q * 16, 16)] = jnp.where(
                        owned, v - lo, dummy)
            _run_pipe(KB, lambda ix: xt_h.at[ix], gi_v, aggp_s, si_v,
                      rows, gsem, ssem)
            return 0

        lax.fori_loop(0, NBLK_A, blk_a, 0)

        b_start = (cid * NSUB + sid) * PB_PER_TILE

        def blk_b(bi, _):
            base = b_start + bi * KB
            pltpu.sync_copy(src_h.at[pl.ds(base, KB)], gi_v)
            pltpu.sync_copy(dst_h.at[pl.ds(base, KB)], si_v)
            for j in range(KB):
                for q in range(B0 // 16):
                    v = gi_v[j, pl.ds(q * 16, 16)]
                    d = si_v[j, pl.ds(q * 16, 16)]
                    dummy = NUM_TEAMS + ((iota16 + q * 16) & DUMMY_MASK)
                    si_v[j, pl.ds(q * 16, 16)] = jnp.where(v >= 0, d, dummy)
                    gi_v[j, pl.ds(q * 16, 16)] = jnp.maximum(v, 0)
            _run_pipe(KB, lambda ix: xp_h.at[ix], gi_v, aggt_s, si_v,
                      rows, gsem, ssem)
            return 0

        lax.fori_loop(0, NBLK_B, blk_b, 0)
        plsc.subcore_barrier()

        def ftchunk(c):
            pltpu.sync_copy(aggt_s.at[pl.ds(c * CHUNK, CHUNK)], zbuf)
            pltpu.sync_copy(zbuf, aggt_out.at[cid].at[pl.ds(c * CHUNK, CHUNK)])

        _strided_chunks(sid, NTCHUNK, ftchunk)

        def fpchunk(c):
            pltpu.sync_copy(aggp_s.at[pl.ds(c * CHUNK, CHUNK)], zbuf)
            pltpu.sync_copy(zbuf, aggp_out.at[pl.ds(lo + c * CHUNK, CHUNK)])

        _strided_chunks(sid, NPCHUNK, fpchunk)

    return k(src2, dst2, x_p, x_t)


def _sc_layer1(src2, dst2, x_p):
    mesh = plsc.VectorSubcoreMesh(core_axis_name="c", subcore_axis_name="s")

    @functools.partial(
        pl.kernel,
        mesh=mesh,
        compiler_params=pltpu.CompilerParams(use_tc_tiling_on_sc=False),
        out_type=jax.ShapeDtypeStruct((NCORES, NUM_TEAMS, FDIM), jnp.float32),
        scratch_types=(
            [pltpu.VMEM_SHARED((AGGT_ROWS, FDIM), jnp.float32),
             pltpu.VMEM((KB_B, B1), jnp.int32),
             pltpu.VMEM((KB_B, B1), jnp.int32)]
            + [pltpu.VMEM((B1, FDIM), jnp.float32) for _ in range(4)]
            + [pltpu.VMEM((CHUNK, FDIM), jnp.float32)]
            + [pltpu.SemaphoreType.DMA for _ in range(8)]
        ),
    )
    def k(src_h, dst_h, xp_h, aggt_out, aggt_s, gi_v, si_v,
          r0, r1, r2, r3, zbuf, g0, g1, g2, g3, s0, s1, s2, s3):
        cid = lax.axis_index("c")
        sid = lax.axis_index("s")
        rows = [r0, r1, r2, r3]
        gsem = [g0, g1, g2, g3]
        ssem = [s0, s1, s2, s3]

        _zero_zbuf(zbuf)
        _strided_chunks(
            sid, NZTCHUNK,
            lambda c: pltpu.sync_copy(zbuf,
                                      aggt_s.at[pl.ds(c * CHUNK, CHUNK)]))
        plsc.subcore_barrier()

        iota16 = lax.iota(jnp.int32, 16)
        start = (cid * NSUB + sid) * L1_PER_TILE

        def blk(bi, _):
            base = start + bi * KB_B
            pltpu.sync_copy(src_h.at[pl.ds(base, KB_B)], gi_v)
            pltpu.sync_copy(dst_h.at[pl.ds(base, KB_B)], si_v)
            for j in range(KB_B):
                for q in range(B1 // 16):
                    v = gi_v[j, pl.ds(q * 16, 16)]
                    d = si_v[j, pl.ds(q * 16, 16)]
                    dummy = NUM_TEAMS + ((iota16 + q * 16) & DUMMY_MASK)
                    si_v[j, pl.ds(q * 16, 16)] = jnp.where(v >= 0, d, dummy)
                    gi_v[j, pl.ds(q * 16, 16)] = jnp.maximum(v, 0)
            _run_pipe(KB_B, lambda ix: xp_h.at[ix], gi_v, aggt_s, si_v,
                      rows, gsem, ssem)
            return 0

        lax.fori_loop(0, NBLK_L1, blk, 0)
        plsc.subcore_barrier()

        def ftchunk(c):
            pltpu.sync_copy(aggt_s.at[pl.ds(c * CHUNK, CHUNK)], zbuf)
            pltpu.sync_copy(zbuf, aggt_out.at[cid].at[pl.ds(c * CHUNK, CHUNK)])

        _strided_chunks(sid, NTCHUNK, ftchunk)

    return k(src2, dst2, x_p)


def _player_update_body(aggp_ref, xp_ref, wr_ref, br_ref, wo_ref, out_ref):
    y = (jnp.dot(aggp_ref[...], wr_ref[...],
                 preferred_element_type=jnp.float32)
         + br_ref[...]
         + jnp.dot(xp_ref[...], wo_ref[...],
                   preferred_element_type=jnp.float32))
    out_ref[...] = jnp.maximum(y, 0.0)


def _player_update(aggp, x_p, Wr, br, Wo):
    blk = 2000
    grid = NUM_PLAYERS // blk
    return pl.pallas_call(
        _player_update_body,
        grid=(grid,),
        in_specs=[
            pl.BlockSpec((blk, FDIM), lambda i: (i, 0)),
            pl.BlockSpec((blk, FDIM), lambda i: (i, 0)),
            pl.BlockSpec((FDIM, FDIM), lambda i: (0, 0)),
            pl.BlockSpec((1, FDIM), lambda i: (0, 0)),
            pl.BlockSpec((FDIM, FDIM), lambda i: (0, 0)),
        ],
        out_specs=pl.BlockSpec((blk, FDIM), lambda i: (i, 0)),
        out_shape=jax.ShapeDtypeStruct((NUM_PLAYERS, FDIM), jnp.float32),
    )(aggp, x_p, Wr, br.reshape(1, FDIM), Wo)


def _final_body(aggt0_ref, aggt1_ref, xt_ref, batch_ref,
                wr0_ref, br0_ref, wo0_ref, wr1_ref, br1_ref, wo1_ref,
                lw_ref, lb_ref, y_ref, xt2_ref, maxs_ref):
    agg0 = aggt0_ref[0] + aggt0_ref[1]
    xt1 = (jnp.dot(agg0, wr0_ref[...], preferred_element_type=jnp.float32)
           + br0_ref[...]
           + jnp.dot(xt_ref[...], wo0_ref[...],
                     preferred_element_type=jnp.float32))
    xt1 = jnp.maximum(xt1, 0.0)
    agg1 = aggt1_ref[0] + aggt1_ref[1]
    x = (jnp.dot(agg1, wr1_ref[...], preferred_element_type=jnp.float32)
         + br1_ref[...]
         + jnp.dot(xt1, wo1_ref[...], preferred_element_type=jnp.float32))
    x = jnp.maximum(x, 0.0)
    xt2_ref[...] = x

    b = batch_ref[...]
    gids = lax.broadcasted_iota(jnp.int32, (NUM_TEAMS, NUM_GRAPHS), 1)
    onehot_t = (b == gids).astype(jnp.float32)
    contract0 = (((0,), (0,)), ((), ()))
    sums = lax.dot_general(onehot_t, x, contract0,
                           preferred_element_type=jnp.float32)
    ones_col = jnp.ones((NUM_TEAMS, 1), jnp.float32)
    counts = lax.dot_general(onehot_t, ones_col, contract0,
                             preferred_element_type=jnp.float32)
    mean = sums / jnp.maximum(counts, 1.0)

    neg_inf = jnp.float32(-jnp.inf)

    def maxrow(g, _):
        mask = b == g
        m = jnp.max(jnp.where(mask, x, neg_inf), axis=0, keepdims=True)
        maxs_ref[pl.ds(g, 1), :] = m
        return 0

    lax.fori_loop(0, NUM_GRAPHS, maxrow, 0)
    maxs = maxs_ref[...]
    maxs = jnp.where(jnp.isfinite(maxs), maxs, 0.0)

    pooled = jnp.concatenate([mean, maxs], axis=-1)
    logits = (jnp.dot(pooled, lw_ref[...], preferred_element_type=jnp.float32)
              + lb_ref[...])
    z = logits - jnp.max(logits, axis=-1, keepdims=True)
    e = jnp.exp(z)
    y_ref[...] = e / jnp.sum(e, axis=-1, keepdims=True)


def _final_stage(aggt0, aggt1, x_t, batch,
                 Wr0, br0, Wo0, Wr1, br1, Wo1, lin_W, lin_b):
    y, xt2, _ = pl.pallas_call(
        _final_body,
        out_shape=[
            jax.ShapeDtypeStruct((NUM_GRAPHS, 32), jnp.float32),
            jax.ShapeDtypeStruct((NUM_TEAMS, FDIM), jnp.float32),
            jax.ShapeDtypeStruct((NUM_GRAPHS, FDIM), jnp.float32),
        ],
    )(aggt0, aggt1, x_t, batch.reshape(NUM_TEAMS, 1),
      Wr0, br0.reshape(1, FDIM), Wo0, Wr1, br1.reshape(1, FDIM), Wo1,
      lin_W, lin_b.reshape(1, 32))
    return y, xt2


def kernel(player_ids, team_ids, edge_src, edge_dst, batch,
           player_emb, team_emb,
           Wr_pt0, br_pt0, Wo_pt0, Wr_tp0, br_tp0, Wo_tp0,
           Wr_pt1, br_pt1, Wo_pt1, Wr_tp1, br_tp1, Wo_tp1,
           lin_W, lin_b):
    x_p = player_emb
    x_t = team_emb

    npad = EDGES_PAD - NUM_EDGES
    src_p = jnp.concatenate(
        [edge_src, jnp.full((npad,), -1, jnp.int32)]).reshape(NSB0, B0)
    dst_p = jnp.concatenate(
        [edge_dst, jnp.zeros((npad,), jnp.int32)]).reshape(NSB0, B0)
    src_p1 = src_p.reshape(NSB1, B1)
    dst_p1 = dst_p.reshape(NSB1, B1)

    aggt0, aggp0 = _sc_layer0(src_p, dst_p, x_p, x_t)
    x_p1 = _player_update(aggp0, x_p, Wr_tp0, br_tp0, Wo_tp0)

    aggt1 = _sc_layer1(src_p1, dst_p1, x_p1)

    y, x_t2 = _final_stage(aggt0, aggt1, x_t, batch,
                           Wr_pt0, br_pt0, Wo_pt0,
                           Wr_pt1, br_pt1, Wo_pt1, lin_W, lin_b)
    return (y, x_t2)


# speedup vs baseline: 2.7430x; 2.7430x over previous
import functools

import jax
import jax.numpy as jnp
from jax import lax
from jax.experimental import pallas as pl
from jax.experimental.pallas import tpu as pltpu
from jax.experimental.pallas import tpu_sc as plsc

NUM_PLAYERS = 50000
NUM_TEAMS = 2000
FDIM = 64
NUM_GRAPHS = 64
NUM_EDGES = 800000

NCORES = 2
NSUB = 16

KB = 16
B0 = 64
B1 = 128

EDGES_PAD = 819200
NSB0 = EDGES_PAD // B0
NSB1 = EDGES_PAD // B1
NBLK_A = NSB0 // (NSUB * KB)
PB_PER_TILE = NSB0 // NCORES // NSUB
NBLK_B = PB_PER_TILE // KB
KB_B = 8
L1_PER_TILE = NSB1 // NCORES // NSUB
NBLK_L1 = L1_PER_TILE // KB_B

P_HALF = NUM_PLAYERS // NCORES
PAD = 160
DUMMY_MASK = 127
CHUNK = 40
AGGP_ROWS = P_HALF + PAD
AGGT_ROWS = NUM_TEAMS + PAD
NPCHUNK = P_HALF // CHUNK
NZPCHUNK = AGGP_ROWS // CHUNK
NTCHUNK = NUM_TEAMS // CHUNK
NZTCHUNK = AGGT_ROWS // CHUNK


def _zero_zbuf(zbuf):
    zero16 = jnp.zeros((16,), jnp.float32)

    def zrow(i, _):
        for j in range(FDIM // 16):
            zbuf[i, pl.ds(j * 16, 16)] = zero16
        return 0

    lax.fori_loop(0, CHUNK, zrow, 0)


def _strided_chunks(sid, nchunk, body):
    def it(i, _):
        c = sid + i * NSUB

        @pl.when(c < nchunk)
        def _():
            body(c)

        return 0

    lax.fori_loop(0, (nchunk + NSUB - 1) // NSUB, it, 0)


def _run_pipe(kb, table_at, gidx, tgt, sidx, rows, gsem, ssem):
    depth = len(rows)
    look = depth - 1
    g = [None] * depth
    pend = [None] * depth
    for j in range(kb + look):
        if j < kb:
            p = j % depth
            if pend[p] is not None:
                pend[p].wait()
                pend[p] = None
            g[p] = pltpu.async_copy(table_at(gidx.at[j]), rows[p], gsem[p])
        if j >= look:
            jj = j - look
            q = jj % depth
            g[q].wait()
            pend[q] = pltpu.async_copy(rows[q], tgt.at[sidx.at[jj]],
                                       ssem[q], add=True)
    for q in range(depth):
        if pend[q] is not None:
            pend[q].wait()


def _sc_layer0(src2, dst2, x_p, x_t):
    mesh = plsc.VectorSubcoreMesh(core_axis_name="c", subcore_axis_name="s")

    @functools.partial(
        pl.kernel,
        mesh=mesh,
        compiler_params=pltpu.CompilerParams(use_tc_tiling_on_sc=False),
        out_type=[
            jax.ShapeDtypeStruct((NCORES, NUM_TEAMS, FDIM), jnp.float32),
            jax.ShapeDtypeStruct((NUM_PLAYERS, FDIM), jnp.float32),
        ],
        scratch_types=(
            [pltpu.VMEM_SHARED((AGGP_ROWS, FDIM), jnp.float32),
             pltpu.VMEM_SHARED((AGGT_ROWS, FDIM), jnp.float32),
             pltpu.VMEM((KB, B0), jnp.int32),
             pltpu.VMEM((KB, B0), jnp.int32)]
            + [pltpu.VMEM((B0, FDIM), jnp.float32) for _ in range(4)]
            + [pltpu.VMEM((CHUNK, FDIM), jnp.float32)]
            + [pltpu.SemaphoreType.DMA for _ in range(8)]
        ),
    )
    def k(src_h, dst_h, xp_h, xt_h, aggt_out, aggp_out,
          aggp_s, aggt_s, gi_v, si_v,
          r0, r1, r2, r3, zbuf, g0, g1, g2, g3, s0, s1, s2, s3):
        cid = lax.axis_index("c")
        sid = lax.axis_index("s")
        lo = cid * P_HALF
        rows = [r0, r1, r2, r3]
        gsem = [g0, g1, g2, g3]
        ssem = [s0, s1, s2, s3]

        _zero_zbuf(zbuf)
        _strided_chunks(
            sid, NZTCHUNK,
            lambda c: pltpu.sync_copy(zbuf,
                                      aggt_s.at[pl.ds(c * CHUNK, CHUNK)]))
        _strided_chunks(
            sid, NZPCHUNK,
            lambda c: pltpu.sync_copy(zbuf,
                                      aggp_s.at[pl.ds(c * CHUNK, CHUNK)]))
        plsc.subcore_barrier()

        iota16 = lax.iota(jnp.int32, 16)

        a_start = ((sid + cid * (NSUB // 2)) % NSUB) * (NBLK_A * KB)

        def blk_a(bi, _):
            base = a_start + bi * KB
            pltpu.sync_copy(dst_h.at[pl.ds(base, KB)], gi_v)
            pltpu.sync_copy(src_h.at[pl.ds(base, KB)], si_v)
            for j in range(KB):
                for q in range(B0 // 16):
                    v = si_v[j, pl.ds(q * 16, 16)]
                    owned = (v >= lo) & (v < lo + P_HALF)
                    dummy = P_HALF + ((iota16 + q * 16 + j * 8)
                                      & DUMMY_MASK)
                    si_v[j, pl.ds(q * 16, 16)] = jnp.where(
                        owned, v - lo, dummy)
            _run_pipe(KB, lambda ix: xt_h.at[ix], gi_v, aggp_s, si_v,
                      rows, gsem, ssem)
            return 0

        lax.fori_loop(0, NBLK_A, blk_a, 0)

        b_start = (cid * NSUB + sid) * PB_PER_TILE

        def blk_b(bi, _):
            base = b_start + bi * KB
            pltpu.sync_copy(src_h.at[pl.ds(base, KB)], gi_v)
            pltpu.sync_copy(dst_h.at[pl.ds(base, KB)], si_v)
            for j in range(KB):
                for q in range(B0 // 16):
                    v = gi_v[j, pl.ds(q * 16, 16)]
                    d = si_v[j, pl.ds(q * 16, 16)]
                    dummy = NUM_TEAMS + ((iota16 + q * 16 + j * 8)
                                         & DUMMY_MASK)
                    srow = iota16 + (q * 16 + j * 64)
                    si_v[j, pl.ds(q * 16, 16)] = jnp.where(v >= 0, d, dummy)
                    gi_v[j, pl.ds(q * 16, 16)] = jnp.where(v >= 0, v, srow)
            _run_pipe(KB, lambda ix: xp_h.at[ix], gi_v, aggt_s, si_v,
                      rows, gsem, ssem)
            return 0

        lax.fori_loop(0, NBLK_B, blk_b, 0)
        plsc.subcore_barrier()

        def ftchunk(c):
            pltpu.sync_copy(aggt_s.at[pl.ds(c * CHUNK, CHUNK)], zbuf)
            pltpu.sync_copy(zbuf, aggt_out.at[cid].at[pl.ds(c * CHUNK, CHUNK)])

        _strided_chunks(sid, NTCHUNK, ftchunk)

        def fpchunk(c):
            pltpu.sync_copy(aggp_s.at[pl.ds(c * CHUNK, CHUNK)], zbuf)
            pltpu.sync_copy(zbuf, aggp_out.at[pl.ds(lo + c * CHUNK, CHUNK)])

        _strided_chunks(sid, NPCHUNK, fpchunk)

    return k(src2, dst2, x_p, x_t)


def _sc_layer1(src2, dst2, x_p):
    mesh = plsc.VectorSubcoreMesh(core_axis_name="c", subcore_axis_name="s")

    @functools.partial(
        pl.kernel,
        mesh=mesh,
        compiler_params=pltpu.CompilerParams(use_tc_tiling_on_sc=False),
        out_type=jax.ShapeDtypeStruct((NCORES, NUM_TEAMS, FDIM), jnp.float32),
        scratch_types=(
            [pltpu.VMEM_SHARED((AGGT_ROWS, FDIM), jnp.float32),
             pltpu.VMEM((KB_B, B1), jnp.int32),
             pltpu.VMEM((KB_B, B1), jnp.int32)]
            + [pltpu.VMEM((B1, FDIM), jnp.float32) for _ in range(4)]
            + [pltpu.VMEM((CHUNK, FDIM), jnp.float32)]
            + [pltpu.SemaphoreType.DMA for _ in range(8)]
        ),
    )
    def k(src_h, dst_h, xp_h, aggt_out, aggt_s, gi_v, si_v,
          r0, r1, r2, r3, zbuf, g0, g1, g2, g3, s0, s1, s2, s3):
        cid = lax.axis_index("c")
        sid = lax.axis_index("s")
        rows = [r0, r1, r2, r3]
        gsem = [g0, g1, g2, g3]
        ssem = [s0, s1, s2, s3]

        _zero_zbuf(zbuf)
        _strided_chunks(
            sid, NZTCHUNK,
            lambda c: pltpu.sync_copy(zbuf,
                                      aggt_s.at[pl.ds(c * CHUNK, CHUNK)]))
        plsc.subcore_barrier()

        iota16 = lax.iota(jnp.int32, 16)
        start = (cid * NSUB + sid) * L1_PER_TILE

        def blk(bi, _):
            base = start + bi * KB_B
            pltpu.sync_copy(src_h.at[pl.ds(base, KB_B)], gi_v)
            pltpu.sync_copy(dst_h.at[pl.ds(base, KB_B)], si_v)
            for j in range(KB_B):
                for q in range(B1 // 16):
                    v = gi_v[j, pl.ds(q * 16, 16)]
                    d = si_v[j, pl.ds(q * 16, 16)]
                    dummy = NUM_TEAMS + ((iota16 + q * 16 + j * 8)
                                         & DUMMY_MASK)
                    srow = iota16 + (q * 16 + j * 128)
                    si_v[j, pl.ds(q * 16, 16)] = jnp.where(v >= 0, d, dummy)
                    gi_v[j, pl.ds(q * 16, 16)] = jnp.where(v >= 0, v, srow)
            _run_pipe(KB_B, lambda ix: xp_h.at[ix], gi_v, aggt_s, si_v,
                      rows, gsem, ssem)
            return 0

        lax.fori_loop(0, NBLK_L1, blk, 0)
        plsc.subcore_barrier()

        def ftchunk(c):
            pltpu.sync_copy(aggt_s.at[pl.ds(c * CHUNK, CHUNK)], zbuf)
            pltpu.sync_copy(zbuf, aggt_out.at[cid].at[pl.ds(c * CHUNK, CHUNK)])

        _strided_chunks(sid, NTCHUNK, ftchunk)

    return k(src2, dst2, x_p)


def _player_update_body(aggp_ref, xp_ref, wr_ref, br_ref, wo_ref, out_ref):
    y = (jnp.dot(aggp_ref[...], wr_ref[...],
                 preferred_element_type=jnp.float32)
         + br_ref[...]
         + jnp.dot(xp_ref[...], wo_ref[...],
                   preferred_element_type=jnp.float32))
    out_ref[...] = jnp.maximum(y, 0.0)


def _player_update(aggp, x_p, Wr, br, Wo):
    blk = 2000
    grid = NUM_PLAYERS // blk
    return pl.pallas_call(
        _player_update_body,
        grid=(grid,),
        in_specs=[
            pl.BlockSpec((blk, FDIM), lambda i: (i, 0)),
            pl.BlockSpec((blk, FDIM), lambda i: (i, 0)),
            pl.BlockSpec((FDIM, FDIM), lambda i: (0, 0)),
            pl.BlockSpec((1, FDIM), lambda i: (0, 0)),
            pl.BlockSpec((FDIM, FDIM), lambda i: (0, 0)),
        ],
        out_specs=pl.BlockSpec((blk, FDIM), lambda i: (i, 0)),
        out_shape=jax.ShapeDtypeStruct((NUM_PLAYERS, FDIM), jnp.float32),
    )(aggp, x_p, Wr, br.reshape(1, FDIM), Wo)


def _final_body(aggt0_ref, aggt1_ref, xt_ref, batch_ref,
                wr0_ref, br0_ref, wo0_ref, wr1_ref, br1_ref, wo1_ref,
                lw_ref, lb_ref, y_ref, xt2_ref, maxs_ref):
    agg0 = aggt0_ref[0] + aggt0_ref[1]
    xt1 = (jnp.dot(agg0, wr0_ref[...], preferred_element_type=jnp.float32)
           + br0_ref[...]
           + jnp.dot(xt_ref[...], wo0_ref[...],
                     preferred_element_type=jnp.float32))
    xt1 = jnp.maximum(xt1, 0.0)
    agg1 = aggt1_ref[0] + aggt1_ref[1]
    x = (jnp.dot(agg1, wr1_ref[...], preferred_element_type=jnp.float32)
         + br1_ref[...]
         + jnp.dot(xt1, wo1_ref[...], preferred_element_type=jnp.float32))
    x = jnp.maximum(x, 0.0)
    xt2_ref[...] = x

    b = batch_ref[...]
    gids = lax.broadcasted_iota(jnp.int32, (NUM_TEAMS, NUM_GRAPHS), 1)
    onehot_t = (b == gids).astype(jnp.float32)
    contract0 = (((0,), (0,)), ((), ()))
    sums = lax.dot_general(onehot_t, x, contract0,
                           preferred_element_type=jnp.float32)
    ones_col = jnp.ones((NUM_TEAMS, 1), jnp.float32)
    counts = lax.dot_general(onehot_t, ones_col, contract0,
                             preferred_element_type=jnp.float32)
    mean = sums / jnp.maximum(counts, 1.0)

    neg_inf = jnp.float32(-jnp.inf)

    def maxrow(g, _):
        mask = b == g
        m = jnp.max(jnp.where(mask, x, neg_inf), axis=0, keepdims=True)
        maxs_ref[pl.ds(g, 1), :] = m
        return 0

    lax.fori_loop(0, NUM_GRAPHS, maxrow, 0)
    maxs = maxs_ref[...]
    maxs = jnp.where(jnp.isfinite(maxs), maxs, 0.0)

    pooled = jnp.concatenate([mean, maxs], axis=-1)
    logits = (jnp.dot(pooled, lw_ref[...], preferred_element_type=jnp.float32)
              + lb_ref[...])
    z = logits - jnp.max(logits, axis=-1, keepdims=True)
    e = jnp.exp(z)
    y_ref[...] = e / jnp.sum(e, axis=-1, keepdims=True)


def _final_stage(aggt0, aggt1, x_t, batch,
                 Wr0, br0, Wo0, Wr1, br1, Wo1, lin_W, lin_b):
    y, xt2, _ = pl.pallas_call(
        _final_body,
        out_shape=[
            jax.ShapeDtypeStruct((NUM_GRAPHS, 32), jnp.float32),
            jax.ShapeDtypeStruct((NUM_TEAMS, FDIM), jnp.float32),
            jax.ShapeDtypeStruct((NUM_GRAPHS, FDIM), jnp.float32),
        ],
    )(aggt0, aggt1, x_t, batch.reshape(NUM_TEAMS, 1),
      Wr0, br0.reshape(1, FDIM), Wo0, Wr1, br1.reshape(1, FDIM), Wo1,
      lin_W, lin_b.reshape(1, 32))
    return y, xt2


def kernel(player_ids, team_ids, edge_src, edge_dst, batch,
           player_emb, team_emb,
           Wr_pt0, br_pt0, Wo_pt0, Wr_tp0, br_tp0, Wo_tp0,
           Wr_pt1, br_pt1, Wo_pt1, Wr_tp1, br_tp1, Wo_tp1,
           lin_W, lin_b):
    x_p = player_emb
    x_t = team_emb

    npad = EDGES_PAD - NUM_EDGES
    src_p = jnp.concatenate(
        [edge_src, jnp.full((npad,), -1, jnp.int32)]).reshape(NSB0, B0)
    dst_p = jnp.concatenate(
        [edge_dst,
         jnp.arange(npad, dtype=jnp.int32) % NUM_TEAMS]).reshape(NSB0, B0)
    src_p1 = src_p.reshape(NSB1, B1)
    dst_p1 = dst_p.reshape(NSB1, B1)

    aggt0, aggp0 = _sc_layer0(src_p, dst_p, x_p, x_t)
    x_p1 = _player_update(aggp0, x_p, Wr_tp0, br_tp0, Wo_tp0)

    aggt1 = _sc_layer1(src_p1, dst_p1, x_p1)

    y, x_t2 = _final_stage(aggt0, aggt1, x_t, batch,
                           Wr_pt0, br_pt0, Wo_pt0,
                           Wr_pt1, br_pt1, Wo_pt1, lin_W, lin_b)
    return (y, x_t2)


# speedup vs baseline: 2.9095x; 1.0607x over previous
import functools

import jax
import jax.numpy as jnp
from jax import lax
from jax.experimental import pallas as pl
from jax.experimental.pallas import tpu as pltpu
from jax.experimental.pallas import tpu_sc as plsc

NUM_PLAYERS = 50000
NUM_TEAMS = 2000
FDIM = 64
NUM_GRAPHS = 64
NUM_EDGES = 800000

NCORES = 2
NSUB = 16

KB = 16
B0 = 64
B1 = 128

EDGES_PAD = 819200
NSB0 = EDGES_PAD // B0
NSB1 = EDGES_PAD // B1
NBLK_A = NSB0 // (NSUB * KB)
PB_PER_TILE = NSB0 // NCORES // NSUB
NBLK_B = PB_PER_TILE // KB
KB_B = 40
L1_PER_TILE = NSB1 // NCORES // NSUB
NBLK_L1 = L1_PER_TILE // KB_B

P_HALF = NUM_PLAYERS // NCORES
PAD = 160
DUMMY_MASK = 127
CHUNK = 40
AGGP_ROWS = P_HALF + PAD
AGGT_ROWS = NUM_TEAMS + PAD
NPCHUNK = P_HALF // CHUNK
NZPCHUNK = AGGP_ROWS // CHUNK
NTCHUNK = NUM_TEAMS // CHUNK
NZTCHUNK = AGGT_ROWS // CHUNK


def _zero_zbuf(zbuf):
    zero16 = jnp.zeros((16,), jnp.float32)

    def zrow(i, _):
        for j in range(FDIM // 16):
            zbuf[i, pl.ds(j * 16, 16)] = zero16
        return 0

    lax.fori_loop(0, CHUNK, zrow, 0)


def _strided_chunks(sid, nchunk, body):
    def it(i, _):
        c = sid + i * NSUB

        @pl.when(c < nchunk)
        def _():
            body(c)

        return 0

    lax.fori_loop(0, (nchunk + NSUB - 1) // NSUB, it, 0)


def _run_pipe(kb, table_at, gidx, tgt, sidx, rows, gsem, ssem):
    depth = len(rows)
    look = depth - 1
    g = [None] * depth
    pend = [None] * depth
    for j in range(kb + look):
        if j < kb:
            p = j % depth
            if pend[p] is not None:
                pend[p].wait()
                pend[p] = None
            g[p] = pltpu.async_copy(table_at(gidx.at[j]), rows[p], gsem[p])
        if j >= look:
            jj = j - look
            q = jj % depth
            g[q].wait()
            pend[q] = pltpu.async_copy(rows[q], tgt.at[sidx.at[jj]],
                                       ssem[q], add=True)
    for q in range(depth):
        if pend[q] is not None:
            pend[q].wait()


def _sc_layer0(src2, dst2, x_p, x_t):
    mesh = plsc.VectorSubcoreMesh(core_axis_name="c", subcore_axis_name="s")

    @functools.partial(
        pl.kernel,
        mesh=mesh,
        compiler_params=pltpu.CompilerParams(use_tc_tiling_on_sc=False),
        out_type=[
            jax.ShapeDtypeStruct((NCORES, NUM_TEAMS, FDIM), jnp.float32),
            jax.ShapeDtypeStruct((NUM_PLAYERS, FDIM), jnp.float32),
        ],
        scratch_types=(
            [pltpu.VMEM_SHARED((AGGP_ROWS, FDIM), jnp.float32),
             pltpu.VMEM_SHARED((AGGT_ROWS, FDIM), jnp.float32),
             pltpu.VMEM((KB, B0), jnp.int32),
             pltpu.VMEM((KB, B0), jnp.int32)]
            + [pltpu.VMEM((B0, FDIM), jnp.float32) for _ in range(4)]
            + [pltpu.VMEM((CHUNK, FDIM), jnp.float32)]
            + [pltpu.SemaphoreType.DMA for _ in range(8)]
        ),
    )
    def k(src_h, dst_h, xp_h, xt_h, aggt_out, aggp_out,
          aggp_s, aggt_s, gi_v, si_v,
          r0, r1, r2, r3, zbuf, g0, g1, g2, g3, s0, s1, s2, s3):
        cid = lax.axis_index("c")
        sid = lax.axis_index("s")
        lo = cid * P_HALF
        rows = [r0, r1, r2, r3]
        gsem = [g0, g1, g2, g3]
        ssem = [s0, s1, s2, s3]

        _zero_zbuf(zbuf)
        _strided_chunks(
            sid, NZTCHUNK,
            lambda c: pltpu.sync_copy(zbuf,
                                      aggt_s.at[pl.ds(c * CHUNK, CHUNK)]))
        _strided_chunks(
            sid, NZPCHUNK,
            lambda c: pltpu.sync_copy(zbuf,
                                      aggp_s.at[pl.ds(c * CHUNK, CHUNK)]))
        plsc.subcore_barrier()

        iota16 = lax.iota(jnp.int32, 16)

        a_start = ((sid + cid * (NSUB // 2)) % NSUB) * (NBLK_A * KB)

        def blk_a(bi, _):
            base = a_start + bi * KB
            pltpu.sync_copy(dst_h.at[pl.ds(base, KB)], gi_v)
            pltpu.sync_copy(src_h.at[pl.ds(base, KB)], si_v)
            for j in range(KB):
                for q in range(B0 // 16):
                    v = si_v[j, pl.ds(q * 16, 16)]
                    owned = (v >= lo) & (v < lo + P_HALF)
                    dummy = P_HALF + ((iota16 + q * 16 + j * 8)
                                      & DUMMY_MASK)
                    si_v[j, pl.ds(q * 16, 16)] = jnp.where(
                        owned, v - lo, dummy)
            _run_pipe(KB, lambda ix: xt_h.at[ix], gi_v, aggp_s, si_v,
                      rows, gsem, ssem)
            return 0

        lax.fori_loop(0, NBLK_A, blk_a, 0)

        b_start = (cid * NSUB + sid) * PB_PER_TILE

        def blk_b(bi, _):
            base = b_start + bi * KB
            pltpu.sync_copy(src_h.at[pl.ds(base, KB)], gi_v)
            pltpu.sync_copy(dst_h.at[pl.ds(base, KB)], si_v)
            for j in range(KB):
                for q in range(B0 // 16):
                    v = gi_v[j, pl.ds(q * 16, 16)]
                    d = si_v[j, pl.ds(q * 16, 16)]
                    dummy = NUM_TEAMS + ((iota16 + q * 16 + j * 8)
                                         & DUMMY_MASK)
                    srow = iota16 + (q * 16 + j * 64)
                    si_v[j, pl.ds(q * 16, 16)] = jnp.where(v >= 0, d, dummy)
                    gi_v[j, pl.ds(q * 16, 16)] = jnp.where(v >= 0, v, srow)
            _run_pipe(KB, lambda ix: xp_h.at[ix], gi_v, aggt_s, si_v,
                      rows, gsem, ssem)
            return 0

        lax.fori_loop(0, NBLK_B, blk_b, 0)
        plsc.subcore_barrier()

        def ftchunk(c):
            pltpu.sync_copy(aggt_s.at[pl.ds(c * CHUNK, CHUNK)], zbuf)
            pltpu.sync_copy(zbuf, aggt_out.at[cid].at[pl.ds(c * CHUNK, CHUNK)])

        _strided_chunks(sid, NTCHUNK, ftchunk)

        def fpchunk(c):
            pltpu.sync_copy(aggp_s.at[pl.ds(c * CHUNK, CHUNK)], zbuf)
            pltpu.sync_copy(zbuf, aggp_out.at[pl.ds(lo + c * CHUNK, CHUNK)])

        _strided_chunks(sid, NPCHUNK, fpchunk)

    return k(src2, dst2, x_p, x_t)


def _sc_layer1(src2, dst2, x_p):
    mesh = plsc.VectorSubcoreMesh(core_axis_name="c", subcore_axis_name="s")

    @functools.partial(
        pl.kernel,
        mesh=mesh,
        compiler_params=pltpu.CompilerParams(use_tc_tiling_on_sc=False),
        out_type=jax.ShapeDtypeStruct((NCORES, NUM_TEAMS, FDIM), jnp.float32),
        scratch_types=(
            [pltpu.VMEM_SHARED((AGGT_ROWS, FDIM), jnp.float32),
             pltpu.VMEM((KB_B, B1), jnp.int32),
             pltpu.VMEM((KB_B, B1), jnp.int32)]
            + [pltpu.VMEM((B1, FDIM), jnp.float32) for _ in range(4)]
            + [pltpu.VMEM((CHUNK, FDIM), jnp.float32)]
            + [pltpu.SemaphoreType.DMA for _ in range(8)]
        ),
    )
    def k(src_h, dst_h, xp_h, aggt_out, aggt_s, gi_v, si_v,
          r0, r1, r2, r3, zbuf, g0, g1, g2, g3, s0, s1, s2, s3):
        cid = lax.axis_index("c")
        sid = lax.axis_index("s")
        rows = [r0, r1, r2, r3]
        gsem = [g0, g1, g2, g3]
        ssem = [s0, s1, s2, s3]

        _zero_zbuf(zbuf)
        _strided_chunks(
            sid, NZTCHUNK,
            lambda c: pltpu.sync_copy(zbuf,
                                      aggt_s.at[pl.ds(c * CHUNK, CHUNK)]))
        plsc.subcore_barrier()

        iota16 = lax.iota(jnp.int32, 16)
        start = (cid * NSUB + sid) * L1_PER_TILE

        def blk(bi, _):
            base = start + bi * KB_B
            pltpu.sync_copy(src_h.at[pl.ds(base, KB_B)], gi_v)
            pltpu.sync_copy(dst_h.at[pl.ds(base, KB_B)], si_v)
            for j in range(KB_B):
                for q in range(B1 // 16):
                    v = gi_v[j, pl.ds(q * 16, 16)]
                    d = si_v[j, pl.ds(q * 16, 16)]
                    dummy = NUM_TEAMS + ((iota16 + q * 16 + j * 8)
                                         & DUMMY_MASK)
                    srow = iota16 + (q * 16 + j * 128)
                    si_v[j, pl.ds(q * 16, 16)] = jnp.where(v >= 0, d, dummy)
                    gi_v[j, pl.ds(q * 16, 16)] = jnp.where(v >= 0, v, srow)
            _run_pipe(KB_B, lambda ix: xp_h.at[ix], gi_v, aggt_s, si_v,
                      rows, gsem, ssem)
            return 0

        lax.fori_loop(0, NBLK_L1, blk, 0)
        plsc.subcore_barrier()

        def ftchunk(c):
            pltpu.sync_copy(aggt_s.at[pl.ds(c * CHUNK, CHUNK)], zbuf)
            pltpu.sync_copy(zbuf, aggt_out.at[cid].at[pl.ds(c * CHUNK, CHUNK)])

        _strided_chunks(sid, NTCHUNK, ftchunk)

    return k(src2, dst2, x_p)


def _player_update_body(aggp_ref, xp_ref, wr_ref, br_ref, wo_ref, out_ref):
    y = (jnp.dot(aggp_ref[...], wr_ref[...],
                 preferred_element_type=jnp.float32)
         + br_ref[...]
         + jnp.dot(xp_ref[...], wo_ref[...],
                   preferred_element_type=jnp.float32))
    out_ref[...] = jnp.maximum(y, 0.0)


def _player_update(aggp, x_p, Wr, br, Wo):
    blk = 2000
    grid = NUM_PLAYERS // blk
    return pl.pallas_call(
        _player_update_body,
        grid=(grid,),
        in_specs=[
            pl.BlockSpec((blk, FDIM), lambda i: (i, 0)),
            pl.BlockSpec((blk, FDIM), lambda i: (i, 0)),
            pl.BlockSpec((FDIM, FDIM), lambda i: (0, 0)),
            pl.BlockSpec((1, FDIM), lambda i: (0, 0)),
            pl.BlockSpec((FDIM, FDIM), lambda i: (0, 0)),
        ],
        out_specs=pl.BlockSpec((blk, FDIM), lambda i: (i, 0)),
        out_shape=jax.ShapeDtypeStruct((NUM_PLAYERS, FDIM), jnp.float32),
    )(aggp, x_p, Wr, br.reshape(1, FDIM), Wo)


def _final_body(aggt0_ref, aggt1_ref, xt_ref, batch_ref,
                wr0_ref, br0_ref, wo0_ref, wr1_ref, br1_ref, wo1_ref,
                lw_ref, lb_ref, y_ref, xt2_ref, maxs_ref):
    agg0 = aggt0_ref[0] + aggt0_ref[1]
    xt1 = (jnp.dot(agg0, wr0_ref[...], preferred_element_type=jnp.float32)
           + br0_ref[...]
           + jnp.dot(xt_ref[...], wo0_ref[...],
                     preferred_element_type=jnp.float32))
    xt1 = jnp.maximum(xt1, 0.0)
    agg1 = aggt1_ref[0] + aggt1_ref[1]
    x = (jnp.dot(agg1, wr1_ref[...], preferred_element_type=jnp.float32)
         + br1_ref[...]
         + jnp.dot(xt1, wo1_ref[...], preferred_element_type=jnp.float32))
    x = jnp.maximum(x, 0.0)
    xt2_ref[...] = x

    b = batch_ref[...]
    gids = lax.broadcasted_iota(jnp.int32, (NUM_TEAMS, NUM_GRAPHS), 1)
    onehot_t = (b == gids).astype(jnp.float32)
    contract0 = (((0,), (0,)), ((), ()))
    sums = lax.dot_general(onehot_t, x, contract0,
                           preferred_element_type=jnp.float32)
    ones_col = jnp.ones((NUM_TEAMS, 1), jnp.float32)
    counts = lax.dot_general(onehot_t, ones_col, contract0,
                             preferred_element_type=jnp.float32)
    mean = sums / jnp.maximum(counts, 1.0)

    neg_inf = jnp.float32(-jnp.inf)

    def maxrow(g, _):
        mask = b == g
        m = jnp.max(jnp.where(mask, x, neg_inf), axis=0, keepdims=True)
        maxs_ref[pl.ds(g, 1), :] = m
        return 0

    lax.fori_loop(0, NUM_GRAPHS, maxrow, 0)
    maxs = maxs_ref[...]
    maxs = jnp.where(jnp.isfinite(maxs), maxs, 0.0)

    pooled = jnp.concatenate([mean, maxs], axis=-1)
    logits = (jnp.dot(pooled, lw_ref[...], preferred_element_type=jnp.float32)
              + lb_ref[...])
    z = logits - jnp.max(logits, axis=-1, keepdims=True)
    e = jnp.exp(z)
    y_ref[...] = e / jnp.sum(e, axis=-1, keepdims=True)


def _final_stage(aggt0, aggt1, x_t, batch,
                 Wr0, br0, Wo0, Wr1, br1, Wo1, lin_W, lin_b):
    y, xt2, _ = pl.pallas_call(
        _final_body,
        out_shape=[
            jax.ShapeDtypeStruct((NUM_GRAPHS, 32), jnp.float32),
            jax.ShapeDtypeStruct((NUM_TEAMS, FDIM), jnp.float32),
            jax.ShapeDtypeStruct((NUM_GRAPHS, FDIM), jnp.float32),
        ],
    )(aggt0, aggt1, x_t, batch.reshape(NUM_TEAMS, 1),
      Wr0, br0.reshape(1, FDIM), Wo0, Wr1, br1.reshape(1, FDIM), Wo1,
      lin_W, lin_b.reshape(1, 32))
    return y, xt2


def kernel(player_ids, team_ids, edge_src, edge_dst, batch,
           player_emb, team_emb,
           Wr_pt0, br_pt0, Wo_pt0, Wr_tp0, br_tp0, Wo_tp0,
           Wr_pt1, br_pt1, Wo_pt1, Wr_tp1, br_tp1, Wo_tp1,
           lin_W, lin_b):
    x_p = player_emb
    x_t = team_emb

    npad = EDGES_PAD - NUM_EDGES
    src_p = jnp.concatenate(
        [edge_src, jnp.full((npad,), -1, jnp.int32)]).reshape(NSB0, B0)
    dst_p = jnp.concatenate(
        [edge_dst,
         jnp.arange(npad, dtype=jnp.int32) % NUM_TEAMS]).reshape(NSB0, B0)
    src_p1 = src_p.reshape(NSB1, B1)
    dst_p1 = dst_p.reshape(NSB1, B1)

    aggt0, aggp0 = _sc_layer0(src_p, dst_p, x_p, x_t)
    x_p1 = _player_update(aggp0, x_p, Wr_tp0, br_tp0, Wo_tp0)

    aggt1 = _sc_layer1(src_p1, dst_p1, x_p1)

    y, x_t2 = _final_stage(aggt0, aggt1, x_t, batch,
                           Wr_pt0, br_pt0, Wo_pt0,
                           Wr_pt1, br_pt1, Wo_pt1, lin_W, lin_b)
    return (y, x_t2)
